# baseline (speedup 1.0000x reference)
import functools

import jax
import jax.numpy as jnp
from jax import lax
from jax.experimental import pallas as pl
from jax.experimental.pallas import tpu as pltpu
from jax.experimental.pallas import tpu_sc as plsc

D = 128
L = 512
CTX = 255
B = 1024
BL = B * L
NC, NS = 2, 16
NW = NC * NS
RW = BL // NW
C = 128
GC = RW // C
LANES = 16
GROUPS = D // LANES


def _sc_body(seq_hbm, tok_hbm, pos_hbm, seg_hbm, out_hbm,
             idx_v, buf_v, bias_v, seg_v, gsem0, gsem1, osem0, osem1):
    wid = lax.axis_index("s") * NC + lax.axis_index("c")
    base = wid * RW

    pltpu.sync_copy(seq_hbm.at[pl.ds(wid * GC, GC)], idx_v)

    pltpu.sync_copy(pos_hbm, bias_v)
    pltpu.sync_copy(seg_hbm, seg_v)

    def seg_add(lo, hi, srow):
        def rbody(l, carry):
            for g in range(GROUPS):
                sl = pl.ds(g * LANES, LANES)
                plsc.addupdate(bias_v.at[l, sl], seg_v[srow, sl])
            return carry
        lax.fori_loop(lo, hi, rbody, 0)

    seg_add(0, 1, 1)
    seg_add(1, 1 + CTX, 2)
    seg_add(1 + CTX, 2 + CTX, 3)
    seg_add(2 + CTX, L, 4)

    gsems = (gsem0, gsem1)
    osems = (osem0, osem1)

    def start_gather(g, slot, sem):
        for k in range(C // LANES):
            iv = idx_v[g, pl.ds(k * LANES, LANES)]
            pltpu.async_copy(tok_hbm.at[iv],
                             buf_v.at[slot, pl.ds(k * LANES, LANES)], sem)

    start_gather(0, 0, gsem0)

    def pair_body(go, carry):
        for b in range(2):
            g = go * 2 + b
            o = 1 - b
            pltpu.make_async_copy(
                tok_hbm.at[idx_v.at[0]], buf_v.at[b], gsems[b]).wait()

            @pl.when(g >= 1)
            def _():
                pltpu.make_async_copy(
                    buf_v.at[o], out_hbm.at[pl.ds(0, C)], osems[o]).wait()

            @pl.when(g + 1 < GC)
            def _():
                start_gather(g + 1, o, gsems[o])

            l0 = (g % (L // C)) * C

            @plsc.parallel_loop(0, C, unroll=4)
            def _(r):
                for grp in range(GROUPS):
                    sl = pl.ds(grp * LANES, LANES)
                    plsc.addupdate(buf_v.at[b, r, sl], bias_v[l0 + r, sl])

            pltpu.async_copy(
                buf_v.at[b], out_hbm.at[pl.ds(base + g * C, C)], osems[b])
        return carry

    lax.fori_loop(0, GC // 2, pair_body, 0)

    pltpu.make_async_copy(
        buf_v.at[(GC - 1) % 2], out_hbm.at[pl.ds(0, C)],
        osems[(GC - 1) % 2]).wait()


_sc_embed = functools.partial(
    pl.kernel,
    out_type=jax.ShapeDtypeStruct((BL, D), jnp.float32),
    mesh=plsc.VectorSubcoreMesh(core_axis_name="c", subcore_axis_name="s",
                                num_cores=NC, num_subcores=NS),
    scratch_types=[
        pltpu.VMEM((GC, C), jnp.int32),
        pltpu.VMEM((2, C, D), jnp.float32),
        pltpu.VMEM((L, D), jnp.float32),
        pltpu.VMEM((5, D), jnp.float32),
        pltpu.SemaphoreType.DMA,
        pltpu.SemaphoreType.DMA,
        pltpu.SemaphoreType.DMA,
        pltpu.SemaphoreType.DMA,
    ],
)(_sc_body)


def kernel(sequence, token_table, pos_table, seg_table):
    seq2d = sequence.reshape(BL // C, C)
    pos_shifted = lax.slice_in_dim(pos_table, 1, L + 1, axis=0)
    out = _sc_embed(seq2d, token_table, pos_shifted, seg_table)
    return out.reshape(B, L, D)


# speedup vs baseline: 1.0756x; 1.0756x over previous
import functools

import jax
import jax.numpy as jnp
from jax import lax
from jax.experimental import pallas as pl
from jax.experimental.pallas import tpu as pltpu
from jax.experimental.pallas import tpu_sc as plsc

D = 128
L = 512
CTX = 255
B = 1024
BL = B * L
NC, NS = 2, 16
NW = NC * NS
HALF = L // 2
BPW = B // (NW // 2)
GC = BPW
C = HALF
LANES = 16
GROUPS = D // LANES


def _sc_body(seq_hbm, tok_hbm, pos_hbm, seg_hbm, out_hbm,
             idx_v, buf_v, bias_v, seg_v, gsem0, gsem1, osem0, osem1):
    wid = lax.axis_index("s") * NC + lax.axis_index("c")
    h = wid % 2
    b0 = (wid // 2) * BPW

    pltpu.sync_copy(seq_hbm.at[pl.ds(h * B + b0, BPW)], idx_v)

    pltpu.sync_copy(pos_hbm.at[pl.ds(h * HALF, HALF)], bias_v)
    pltpu.sync_copy(seg_hbm, seg_v)
    s_first = 1 + 2 * h
    s_rest = 2 + 2 * h
    for grp in range(GROUPS):
        sl = pl.ds(grp * LANES, LANES)
        plsc.addupdate(bias_v.at[0, sl], seg_v[s_first, sl])

    @plsc.parallel_loop(1, HALF, unroll=4)
    def _(r):
        for grp in range(GROUPS):
            sl = pl.ds(grp * LANES, LANES)
            plsc.addupdate(bias_v.at[r, sl], seg_v[s_rest, sl])

    gsems = (gsem0, gsem1)
    osems = (osem0, osem1)

    def start_gather(g, slot, sem):
        for k in range(C // LANES):
            iv = idx_v[g, pl.ds(k * LANES, LANES)]
            pltpu.async_copy(tok_hbm.at[iv],
                             buf_v.at[slot, pl.ds(k * LANES, LANES)], sem)

    start_gather(0, 0, gsem0)

    def pair_body(go, carry):
        for b in range(2):
            g = go * 2 + b
            o = 1 - b
            pltpu.make_async_copy(
                tok_hbm.at[idx_v.at[0]], buf_v.at[b], gsems[b]).wait()

            @pl.when(g >= 1)
            def _():
                pltpu.make_async_copy(
                    buf_v.at[o], out_hbm.at[pl.ds(0, C)], osems[o]).wait()

            @pl.when(g + 1 < GC)
            def _():
                start_gather(g + 1, o, gsems[o])

            @plsc.parallel_loop(0, C, unroll=4)
            def _(r):
                for grp in range(GROUPS):
                    sl = pl.ds(grp * LANES, LANES)
                    plsc.addupdate(buf_v.at[b, r, sl], bias_v[r, sl])

            row0 = (b0 + g) * L + h * HALF
            pltpu.async_copy(
                buf_v.at[b], out_hbm.at[pl.ds(row0, C)], osems[b])
        return carry

    lax.fori_loop(0, GC // 2, pair_body, 0)

    pltpu.make_async_copy(
        buf_v.at[(GC - 1) % 2], out_hbm.at[pl.ds(0, C)],
        osems[(GC - 1) % 2]).wait()


_sc_embed = functools.partial(
    pl.kernel,
    out_type=jax.ShapeDtypeStruct((BL, D), jnp.float32),
    mesh=plsc.VectorSubcoreMesh(core_axis_name="c", subcore_axis_name="s",
                                num_cores=NC, num_subcores=NS),
    scratch_types=[
        pltpu.VMEM((GC, C), jnp.int32),
        pltpu.VMEM((2, C, D), jnp.float32),
        pltpu.VMEM((HALF, D), jnp.float32),
        pltpu.VMEM((5, D), jnp.float32),
        pltpu.SemaphoreType.DMA,
        pltpu.SemaphoreType.DMA,
        pltpu.SemaphoreType.DMA,
        pltpu.SemaphoreType.DMA,
    ],
)(_sc_body)


def kernel(sequence, token_table, pos_table, seg_table):
    seq2 = sequence.reshape(B, 2, HALF).transpose(1, 0, 2).reshape(2 * B, HALF)
    pos_shifted = lax.slice_in_dim(pos_table, 1, L + 1, axis=0)
    out = _sc_embed(seq2, token_table, pos_shifted, seg_table)
    return out.reshape(B, L, D)


# speedup vs baseline: 1.0784x; 1.0026x over previous
import functools

import jax
import jax.numpy as jnp
from jax import lax
from jax.experimental import pallas as pl
from jax.experimental.pallas import tpu as pltpu
from jax.experimental.pallas import tpu_sc as plsc

D = 128
L = 512
CTX = 255
B = 1024
BL = B * L
NC, NS = 2, 16
NW = NC * NS
HALF = L // 2
BPW = B // (NW // 2)
C = 128
GC = BPW * HALF // C
NBUF = 2
LANES = 16
GROUPS = D // LANES


def _sc_body(seq_hbm, tok_hbm, pos_hbm, seg_hbm, out_hbm,
             idx_v, buf_v, bias_v, seg_v, gsems, osems):
    wid = lax.axis_index("s") * NC + lax.axis_index("c")
    h = wid % 2
    b0 = (wid // 2) * BPW

    pltpu.sync_copy(seq_hbm.at[pl.ds(h * B + b0, BPW)], idx_v)

    pltpu.sync_copy(pos_hbm.at[pl.ds(h * HALF, HALF)], bias_v)
    pltpu.sync_copy(seg_hbm, seg_v)
    s_first = 1 + 2 * h
    s_rest = 2 + 2 * h
    for grp in range(GROUPS):
        sl = pl.ds(grp * LANES, LANES)
        plsc.addupdate(bias_v.at[0, sl], seg_v[s_first, sl])

    @plsc.parallel_loop(1, HALF, unroll=4)
    def _(r):
        for grp in range(GROUPS):
            sl = pl.ds(grp * LANES, LANES)
            plsc.addupdate(bias_v.at[r, sl], seg_v[s_rest, sl])

    def start_gather(g, slot, sem):
        pltpu.async_copy(
            tok_hbm.at[idx_v.at[g // 2, pl.ds((g % 2) * C, C)]],
            buf_v.at[slot], sem)

    start_gather(0, 0, gsems[0])

    def ring_body(go, carry):
        for s in range(NBUF):
            g = go * NBUF + s
            pltpu.make_async_copy(
                tok_hbm.at[idx_v.at[0, pl.ds(0, C)]], buf_v.at[s],
                gsems[s]).wait()

            @pl.when(g >= 1)
            def _():
                pltpu.make_async_copy(
                    buf_v.at[1 - s], out_hbm.at[pl.ds(0, C)],
                    osems[1 - s]).wait()

            @pl.when(g + 1 < GC)
            def _():
                start_gather(g + 1, 1 - s, gsems[1 - s])

            @plsc.parallel_loop(0, C, unroll=4)
            def _(r):
                for grp in range(GROUPS):
                    sl = pl.ds(grp * LANES, LANES)
                    plsc.addupdate(buf_v.at[s, r, sl], bias_v[s * C + r, sl])

            row0 = (b0 + g // 2) * L + h * HALF + s * C
            pltpu.async_copy(
                buf_v.at[s], out_hbm.at[pl.ds(row0, C)], osems[s])
        return carry

    lax.fori_loop(0, GC // NBUF, ring_body, 0)

    pltpu.make_async_copy(
        buf_v.at[1], out_hbm.at[pl.ds(0, C)], osems[1]).wait()


_sc_embed = functools.partial(
    pl.kernel,
    out_type=jax.ShapeDtypeStruct((BL, D), jnp.float32),
    mesh=plsc.VectorSubcoreMesh(core_axis_name="c", subcore_axis_name="s",
                                num_cores=NC, num_subcores=NS),
    scratch_types=[
        pltpu.VMEM((BPW, HALF), jnp.int32),
        pltpu.VMEM((NBUF, C, D), jnp.float32),
        pltpu.VMEM((HALF, D), jnp.float32),
        pltpu.VMEM((5, D), jnp.float32),
        [pltpu.SemaphoreType.DMA] * NBUF,
        [pltpu.SemaphoreType.DMA] * NBUF,
    ],
)(_sc_body)


def kernel(sequence, token_table, pos_table, seg_table):
    seq2 = sequence.reshape(B, 2, HALF).transpose(1, 0, 2).reshape(2 * B, HALF)
    pos_shifted = lax.slice_in_dim(pos_table, 1, L + 1, axis=0)
    out = _sc_embed(seq2, token_table, pos_shifted, seg_table)
    return out.reshape(B, L, D)
